# baseline (speedup 1.0000x reference)
import functools

import jax
import jax.numpy as jnp
from jax import lax
from jax.experimental import pallas as pl
from jax.experimental.pallas import tpu as pltpu
from jax.experimental.pallas import tpu_sc as plsc

N = 10000
D = 128
H = 4
NCONV = 2

NCORE = 2
NSUB = 16
NW = NCORE * NSUB
LANES = 16
CHUNK = 128
NP = 10240
ROWS_PER_SUB = NP // NSUB

_mesh = plsc.VectorSubcoreMesh(core_axis_name="c", subcore_axis_name="s")
_sc_params = pltpu.CompilerParams(needs_layout_passes=False)


def _pad_edge_arrays(src, dst):
    e = src.shape[0]
    nch = -(-e // (NW * CHUNK))
    nch += nch % 2
    ep = NW * CHUNK * nch
    pad = ep - e
    src_p = jnp.concatenate([src, jnp.full((pad,), N, jnp.int32)])
    dst_p = jnp.concatenate([dst, jnp.full((pad,), N, jnp.int32)])
    return (src_p.reshape(NW, nch, CHUNK), dst_p.reshape(NW, nch, CHUNK), nch)


def _sc_degree(dst_p, nch):
    @functools.partial(
        pl.kernel, mesh=_mesh, compiler_params=_sc_params,
        out_type=jax.ShapeDtypeStruct((NW, NP), jnp.float32),
        scratch_types=[
            pltpu.VMEM((nch, CHUNK), jnp.int32),
            pltpu.VMEM((NP,), jnp.float32),
            pltpu.SemaphoreType.DMA,
        ],
    )
    def k(dst_hbm, out_hbm, idx_v, deg_v, sem):
        c = lax.axis_index("c")
        s = lax.axis_index("s")
        w = s * NCORE + c
        pltpu.sync_copy(dst_hbm.at[w], idx_v)

        @pl.loop(0, NP, step=LANES)
        def _zero(i):
            deg_v[pl.ds(i, LANES)] = jnp.zeros((LANES,), jnp.float32)

        ones = jnp.ones((LANES,), jnp.float32)

        @pl.loop(0, nch)
        def _edges(j):
            for q in range(CHUNK // LANES):
                dvec = idx_v[j, pl.ds(q * LANES, LANES)]
                plsc.addupdate_scatter(deg_v, [dvec], ones)

        pltpu.sync_copy(deg_v, out_hbm.at[w])

    return k(dst_p)


def _sc_segsum(h_pad, src_p, dst_p, nch):
    @functools.partial(
        pl.kernel, mesh=_mesh, compiler_params=_sc_params,
        out_type=jax.ShapeDtypeStruct((NCORE, NSUB, ROWS_PER_SUB, D),
                                      jnp.float32),
        scratch_types=[
            pltpu.VMEM((nch, CHUNK), jnp.int32),
            pltpu.VMEM((2, CHUNK), jnp.int32),
            pltpu.VMEM((2, CHUNK, D), jnp.float32),
            pltpu.VMEM_SHARED((NP, D), jnp.float32),
            pltpu.SemaphoreType.DMA,
            pltpu.SemaphoreType.DMA,
            pltpu.SemaphoreType.DMA,
            pltpu.SemaphoreType.DMA,
            pltpu.SemaphoreType.DMA,
            pltpu.SemaphoreType.DMA,
        ],
    )
    def k(h_hbm, src_hbm, dst_hbm, out_hbm, src_v, didx_v, rows_v,
          acc_sh, g0, g1, i0, i1, s0, s1):
        c = lax.axis_index("c")
        s = lax.axis_index("s")
        w = s * NCORE + c
        gsem = (g0, g1)
        isem = (i0, i1)
        ssem = (s0, s1)
        pltpu.sync_copy(src_hbm.at[w], src_v)

        @pl.loop(0, CHUNK)
        def _zb(i):
            for q in range(D // LANES):
                rows_v[0, i, pl.ds(q * LANES, LANES)] = jnp.zeros(
                    (LANES,), jnp.float32)

        base = s * ROWS_PER_SUB

        @pl.loop(0, ROWS_PER_SUB, step=CHUNK)
        def _za(r):
            pltpu.sync_copy(rows_v.at[0], acc_sh.at[pl.ds(base + r, CHUNK)])

        plsc.subcore_barrier()

        pltpu.async_copy(dst_hbm.at[w, 0], didx_v.at[0], isem[0])
        pltpu.async_copy(h_hbm.at[src_v.at[0]], rows_v.at[0], gsem[0])

        @pl.loop(0, nch, step=2)
        def _edges(j):
            for b in range(2):
                jj = j + b
                nb = b ^ 1

                @pl.when(jj + 1 < nch)
                def _prefetch():
                    @pl.when(jj >= 1)
                    def _wait_sc():
                        pltpu.make_async_copy(
                            rows_v.at[nb], acc_sh.at[didx_v.at[nb]],
                            ssem[nb]).wait()
                    pltpu.async_copy(dst_hbm.at[w, jj + 1], didx_v.at[nb],
                                     isem[nb])
                    pltpu.async_copy(h_hbm.at[src_v.at[jj + 1]],
                                     rows_v.at[nb], gsem[nb])

                pltpu.make_async_copy(h_hbm.at[src_v.at[jj]], rows_v.at[b],
                                      gsem[b]).wait()
                pltpu.make_async_copy(dst_hbm.at[w, jj], didx_v.at[b],
                                      isem[b]).wait()
                pltpu.async_copy(rows_v.at[b], acc_sh.at[didx_v.at[b]],
                                 ssem[b], add=True)

        pltpu.make_async_copy(rows_v.at[0], acc_sh.at[didx_v.at[0]],
                              ssem[0]).wait()
        pltpu.make_async_copy(rows_v.at[1], acc_sh.at[didx_v.at[1]],
                              ssem[1]).wait()
        plsc.subcore_barrier()
        pltpu.sync_copy(acc_sh.at[pl.ds(base, ROWS_PER_SUB)], out_hbm.at[c, s])

    out = k(h_pad, src_p, dst_p)
    return out.reshape(NCORE, NP, D)


def _sc_attention(z_pad, es_pad, ed_pad, src_p, dst_p, nch):
    @functools.partial(
        pl.kernel, mesh=_mesh, compiler_params=_sc_params,
        out_type=(
            jax.ShapeDtypeStruct((NCORE, NSUB, ROWS_PER_SUB, D), jnp.float32),
            jax.ShapeDtypeStruct((NCORE, NSUB, ROWS_PER_SUB), jnp.float32),
        ),
        scratch_types=[
            pltpu.VMEM((nch, CHUNK), jnp.int32),
            pltpu.VMEM((2, CHUNK), jnp.int32),
            pltpu.VMEM((CHUNK,), jnp.float32),
            pltpu.VMEM((CHUNK,), jnp.float32),
            pltpu.VMEM((CHUNK,), jnp.float32),
            pltpu.VMEM((CHUNK,), jnp.float32),
            pltpu.VMEM((2, CHUNK, D), jnp.float32),
            pltpu.VMEM_SHARED((NP, D), jnp.float32),
            pltpu.VMEM_SHARED((NP,), jnp.float32),
        ] + [pltpu.SemaphoreType.DMA] * 10,
    )
    def k(z_hbm, es_hbm, ed_hbm, src_hbm, dst_hbm, num_hbm, den_hbm,
          src_v, didx_v, wes0, wes1, wed0, wed1, rows_v, acc_sh, den_sh,
          g0, g1, e0, e1, f0, f1, s0, s1, d0, d1):
        c = lax.axis_index("c")
        s = lax.axis_index("s")
        w = s * NCORE + c
        gsem = (g0, g1)
        esem = (e0, e1)
        fsem = (f0, f1)
        ssem = (s0, s1)
        dsem = (d0, d1)
        wes = (wes0, wes1)
        wed = (wed0, wed1)
        pltpu.sync_copy(src_hbm.at[w], src_v)

        @pl.loop(0, CHUNK)
        def _zb(i):
            for q in range(D // LANES):
                rows_v[0, i, pl.ds(q * LANES, LANES)] = jnp.zeros(
                    (LANES,), jnp.float32)

        @pl.loop(0, CHUNK, step=LANES)
        def _zw(i):
            wes0[pl.ds(i, LANES)] = jnp.zeros((LANES,), jnp.float32)

        base = s * ROWS_PER_SUB

        @pl.loop(0, ROWS_PER_SUB, step=CHUNK)
        def _za(r):
            pltpu.sync_copy(rows_v.at[0], acc_sh.at[pl.ds(base + r, CHUNK)])
            pltpu.sync_copy(wes0, den_sh.at[pl.ds(base + r, CHUNK)])

        plsc.subcore_barrier()

        pltpu.sync_copy(dst_hbm.at[w, 0], didx_v.at[0])
        pltpu.async_copy(z_hbm.at[src_v.at[0]], rows_v.at[0], gsem[0])
        pltpu.async_copy(es_hbm.at[src_v.at[0]], wes[0], esem[0])
        pltpu.async_copy(ed_hbm.at[didx_v.at[0]], wed[0], fsem[0])

        @pl.loop(0, nch, step=2)
        def _edges(j):
            for b in range(2):
                jj = j + b
                nb = b ^ 1

                @pl.when(jj + 1 < nch)
                def _prefetch():
                    @pl.when(jj >= 1)
                    def _wait_reuse():
                        pltpu.make_async_copy(
                            rows_v.at[nb], acc_sh.at[didx_v.at[nb]],
                            ssem[nb]).wait()
                        pltpu.make_async_copy(
                            wes[nb], den_sh.at[didx_v.at[nb]],
                            dsem[nb]).wait()
                    pltpu.sync_copy(dst_hbm.at[w, jj + 1], didx_v.at[nb])
                    pltpu.async_copy(z_hbm.at[src_v.at[jj + 1]],
                                     rows_v.at[nb], gsem[nb])
                    pltpu.async_copy(es_hbm.at[src_v.at[jj + 1]], wes[nb],
                                     esem[nb])
                    pltpu.async_copy(ed_hbm.at[didx_v.at[nb]], wed[nb],
                                     fsem[nb])

                pltpu.make_async_copy(es_hbm.at[src_v.at[jj]], wes[b],
                                      esem[b]).wait()
                pltpu.make_async_copy(ed_hbm.at[didx_v.at[b]], wed[b],
                                      fsem[b]).wait()
                for q in range(CHUNK // LANES):
                    sl = pl.ds(q * LANES, LANES)
                    e = wes[b][sl] + wed[b][sl]
                    e = jnp.where(e > 0, e, 0.2 * e)
                    wes[b][sl] = jnp.exp(e)
                pltpu.async_copy(wes[b], den_sh.at[didx_v.at[b]], dsem[b],
                                 add=True)

                pltpu.make_async_copy(z_hbm.at[src_v.at[jj]], rows_v.at[b],
                                      gsem[b]).wait()

                @pl.loop(0, CHUNK)
                def _scale(ei):
                    wb = plsc.load_gather(wes[b],
                                          [jnp.broadcast_to(ei, (LANES,))])
                    for q in range(D // LANES):
                        sl = pl.ds(q * LANES, LANES)
                        rows_v[b, ei, sl] = rows_v[b, ei, sl] * wb

                pltpu.async_copy(rows_v.at[b], acc_sh.at[didx_v.at[b]],
                                 ssem[b], add=True)

        for b in range(2):
            pltpu.make_async_copy(rows_v.at[b], acc_sh.at[didx_v.at[b]],
                                  ssem[b]).wait()
            pltpu.make_async_copy(wes[b], den_sh.at[didx_v.at[b]],
                                  dsem[b]).wait()
        plsc.subcore_barrier()
        pltpu.sync_copy(acc_sh.at[pl.ds(base, ROWS_PER_SUB)], num_hbm.at[c, s])
        pltpu.sync_copy(den_sh.at[pl.ds(base, ROWS_PER_SUB)],
                        den_hbm.at[c, s])

    num, den = k(z_pad, es_pad, ed_pad, src_p, dst_p)
    return num.reshape(NCORE, NP, D), den.reshape(NCORE, NP)


BN = 2048


def _rows(pid):
    return pid * BN + lax.broadcasted_iota(jnp.int32, (BN, 1), 0)


def _ln_rows(t, g, b):
    mu = jnp.mean(t, axis=-1, keepdims=True)
    v = jnp.mean((t - mu) ** 2, axis=-1, keepdims=True)
    return (t - mu) / jnp.sqrt(v + 1e-5) * g + b


def _tc_rs(deg_t):
    def body(d_ref, o_ref):
        o_ref[...] = lax.rsqrt(
            jnp.sum(d_ref[...], axis=1, keepdims=True) + 1.0)

    return pl.pallas_call(
        body, out_shape=jax.ShapeDtypeStruct((NP, 1), jnp.float32))(deg_t)


def _tc_prescale(xp, W, b, rs):
    def body(x_ref, w_ref, b_ref, rs_ref, o_ref):
        h = jnp.dot(x_ref[...], w_ref[...],
                    preferred_element_type=jnp.float32) + b_ref[...]
        h = h * rs_ref[...]
        o_ref[...] = jnp.where(_rows(pl.program_id(0)) < N, h, 0.0)

    return pl.pallas_call(
        body,
        grid=(NP // BN,),
        in_specs=[pl.BlockSpec((BN, D), lambda i: (i, 0)),
                  pl.BlockSpec((D, D), lambda i: (0, 0)),
                  pl.BlockSpec((1, D), lambda i: (0, 0)),
                  pl.BlockSpec((BN, 1), lambda i: (i, 0))],
        out_specs=pl.BlockSpec((BN, D), lambda i: (i, 0)),
        out_shape=jax.ShapeDtypeStruct((NP, D), jnp.float32),
    )(xp, W, b.reshape(1, D), rs)


def _tc_conv_mid(p, rs, g, bb, W, b):
    def body(p_ref, rs_ref, g_ref, bb_ref, w_ref, b_ref, o_ref):
        rsb = rs_ref[...]
        t = (p_ref[0] + p_ref[1]) * rsb
        y = jax.nn.relu(_ln_rows(t, g_ref[...], bb_ref[...]))
        h = (jnp.dot(y, w_ref[...], preferred_element_type=jnp.float32)
             + b_ref[...]) * rsb
        o_ref[...] = jnp.where(_rows(pl.program_id(0)) < N, h, 0.0)

    return pl.pallas_call(
        body,
        grid=(NP // BN,),
        in_specs=[pl.BlockSpec((NCORE, BN, D), lambda i: (0, i, 0)),
                  pl.BlockSpec((BN, 1), lambda i: (i, 0)),
                  pl.BlockSpec((1, D), lambda i: (0, 0)),
                  pl.BlockSpec((1, D), lambda i: (0, 0)),
                  pl.BlockSpec((D, D), lambda i: (0, 0)),
                  pl.BlockSpec((1, D), lambda i: (0, 0))],
        out_specs=pl.BlockSpec((BN, D), lambda i: (i, 0)),
        out_shape=jax.ShapeDtypeStruct((NP, D), jnp.float32),
    )(p, rs, g.reshape(1, D), bb.reshape(1, D), W, b.reshape(1, D))


def _tc_att_prep(p, rs, g, bb, W_att, a_cat):
    def body(p_ref, rs_ref, g_ref, bb_ref, watt_ref, a_ref,
             z0_ref, z1_ref, z2_ref, z3_ref, e8_ref):
        t = (p_ref[0] + p_ref[1]) * rs_ref[...]
        y = jax.nn.relu(_ln_rows(t, g_ref[...], bb_ref[...]))
        y = jnp.where(_rows(pl.program_id(0)) < N, y, 0.0)
        z_refs = [z0_ref, z1_ref, z2_ref, z3_ref]
        cols = []
        for h in range(H):
            z = jnp.dot(y, watt_ref[h], preferred_element_type=jnp.float32)
            z_refs[h][...] = z
            cols.append(jnp.dot(z, a_ref[h],
                                preferred_element_type=jnp.float32))
        e8_ref[...] = jnp.concatenate(cols, axis=-1)

    zs = jax.ShapeDtypeStruct((NP, D), jnp.float32)
    return pl.pallas_call(
        body,
        grid=(NP // BN,),
        in_specs=[pl.BlockSpec((NCORE, BN, D), lambda i: (0, i, 0)),
                  pl.BlockSpec((BN, 1), lambda i: (i, 0)),
                  pl.BlockSpec((1, D), lambda i: (0, 0)),
                  pl.BlockSpec((1, D), lambda i: (0, 0)),
                  pl.BlockSpec((H, D, D), lambda i: (0, 0, 0)),
                  pl.BlockSpec((H, D, 2), lambda i: (0, 0, 0))],
        out_specs=[pl.BlockSpec((BN, D), lambda i: (i, 0))] * H
        + [pl.BlockSpec((BN, 2 * H), lambda i: (i, 0))],
        out_shape=[zs] * H
        + [jax.ShapeDtypeStruct((NP, 2 * H), jnp.float32)],
    )(p, rs, g.reshape(1, D), bb.reshape(1, D), W_att, a_cat)


def _tc_final(nums, den_ts, g2, b2, Wp, bp):
    def body(n0, n1, n2, n3, d0, d1, d2, d3, g_ref, bb_ref, w_ref, b_ref,
             o_ref):
        hs = []
        for n_ref, d_ref in zip((n0, n1, n2, n3), (d0, d1, d2, d3)):
            den = jnp.sum(d_ref[...], axis=1, keepdims=True) + 1e-9
            hs.append((n_ref[0] + n_ref[1]) / den)
        hcat = jnp.concatenate(hs, axis=-1)
        y = _ln_rows(hcat, g_ref[...], bb_ref[...])
        y = jnp.dot(y, w_ref[...], preferred_element_type=jnp.float32)
        o_ref[...] = jax.nn.relu(y + b_ref[...])

    hd = 2 * H * D
    return pl.pallas_call(
        body,
        grid=(NP // BN,),
        in_specs=[pl.BlockSpec((NCORE, BN, D), lambda i: (0, i, 0))] * H
        + [pl.BlockSpec((BN, NCORE), lambda i: (i, 0))] * H
        + [pl.BlockSpec((1, H * D), lambda i: (0, 0)),
           pl.BlockSpec((1, H * D), lambda i: (0, 0)),
           pl.BlockSpec((H * D, D), lambda i: (0, 0)),
           pl.BlockSpec((1, D), lambda i: (0, 0))],
        out_specs=pl.BlockSpec((BN, D), lambda i: (i, 0)),
        out_shape=jax.ShapeDtypeStruct((NP, D), jnp.float32),
    )(*nums, *den_ts, g2.reshape(1, H * D), b2.reshape(1, H * D), Wp,
      bp.reshape(1, D))


def kernel(x, edge_index, W_conv, b_conv, ln_g, ln_b, ln2_g, ln2_b, W_att,
           a_src, a_dst, W_proj, b_proj):
    src = edge_index[0]
    dst = edge_index[1]
    src_p, dst_p, nch = _pad_edge_arrays(src, dst)

    deg_part = _sc_degree(dst_p, nch)
    rs = _tc_rs(deg_part.T)

    xp = jnp.pad(x, ((0, NP - N), (0, 0)))
    h = _tc_prescale(xp, W_conv[0], b_conv[0], rs)
    p = _sc_segsum(h, src_p, dst_p, nch)
    h = _tc_conv_mid(p, rs, ln_g, ln_b, W_conv[1], b_conv[1])
    p = _sc_segsum(h, src_p, dst_p, nch)

    a_cat = jnp.stack([a_src, a_dst], axis=-1)
    *zs, e8 = _tc_att_prep(p, rs, ln_g, ln_b, W_att, a_cat)

    nums, den_ts = [], []
    for hh in range(H):
        num, den = _sc_attention(zs[hh], e8[:, 2 * hh], e8[:, 2 * hh + 1],
                                 src_p, dst_p, nch)
        nums.append(num)
        den_ts.append(den.T)
    out = _tc_final(nums, den_ts, ln2_g, ln2_b, W_proj, b_proj)
    return out[:N]


# speedup vs baseline: 1.1093x; 1.1093x over previous
import functools

import jax
import jax.numpy as jnp
from jax import lax
from jax.experimental import pallas as pl
from jax.experimental.pallas import tpu as pltpu
from jax.experimental.pallas import tpu_sc as plsc

N = 10000
D = 128
H = 4
NCONV = 2

NCORE = 2
NSUB = 16
NW = NCORE * NSUB
LANES = 16
CHUNK = 128
NP = 10240
ROWS_PER_SUB = NP // NSUB

_mesh = plsc.VectorSubcoreMesh(core_axis_name="c", subcore_axis_name="s")
_sc_params = pltpu.CompilerParams(needs_layout_passes=False)


SLOW_CORE = 1
F_SLOW_CONV = 0.22
F_SLOW_ATT = 0.30


def _split_counts(e, f_slow):
    total = -(-e // CHUNK)
    t_slow = max(8, 8 * int(round(total * f_slow / (8 * NSUB))))
    t_fast = 8 * (-(-(total - NSUB * t_slow) // (8 * NSUB)))
    while t_fast > 120:
        t_slow += 8
        t_fast = 8 * (-(-(total - NSUB * t_slow) // (8 * NSUB)))
    if SLOW_CORE == 0:
        return t_slow, t_fast
    return t_fast, t_slow


def _pad_edge_arrays(src, dst, tc0, tc1):
    e = src.shape[0]
    tot = NSUB * (tc0 + tc1) + abs(tc0 - tc1)
    pad = tot * CHUNK - e
    src_p = jnp.concatenate([src, jnp.full((pad,), N, jnp.int32)])
    dst_p = jnp.concatenate([dst, jnp.full((pad,), N, jnp.int32)])
    return src_p.reshape(tot, CHUNK), dst_p.reshape(tot, CHUNK)


def _tile_range(c, s, tc0, tc1):
    cnt = jnp.where(c == 0, tc0, tc1)
    start = jnp.where(c == 0, s * tc0, NSUB * tc0 + s * tc1)
    return pl.multiple_of(start, 8), cnt


def _sc_degree(dst_p, tc0, tc1):
    tmax = max(tc0, tc1)

    @functools.partial(
        pl.kernel, mesh=_mesh, compiler_params=_sc_params,
        out_type=jax.ShapeDtypeStruct((NW, NP), jnp.float32),
        scratch_types=[
            pltpu.VMEM((tmax, CHUNK), jnp.int32),
            pltpu.VMEM((NP,), jnp.float32),
            pltpu.SemaphoreType.DMA,
        ],
    )
    def k(dst_hbm, out_hbm, idx_v, deg_v, sem):
        c = lax.axis_index("c")
        s = lax.axis_index("s")
        w = s * NCORE + c
        start, cnt = _tile_range(c, s, tc0, tc1)
        pltpu.sync_copy(dst_hbm.at[pl.ds(start, tmax)], idx_v)

        @pl.loop(0, NP, step=LANES)
        def _zero(i):
            deg_v[pl.ds(i, LANES)] = jnp.zeros((LANES,), jnp.float32)

        ones = jnp.ones((LANES,), jnp.float32)

        @pl.loop(0, cnt)
        def _edges(j):
            for q in range(CHUNK // LANES):
                dvec = idx_v[j, pl.ds(q * LANES, LANES)]
                plsc.addupdate_scatter(deg_v, [dvec], ones)

        pltpu.sync_copy(deg_v, out_hbm.at[w])

    return k(dst_p)


def _sc_segsum(h_pad, src_p, dst_p, tc0, tc1):
    tmax = max(tc0, tc1)

    @functools.partial(
        pl.kernel, mesh=_mesh, compiler_params=_sc_params,
        out_type=jax.ShapeDtypeStruct((NCORE, NSUB, ROWS_PER_SUB, D),
                                      jnp.float32),
        scratch_types=[
            pltpu.VMEM((tmax, CHUNK), jnp.int32),
            pltpu.VMEM((2, CHUNK), jnp.int32),
            pltpu.VMEM((2, CHUNK, D), jnp.float32),
            pltpu.VMEM_SHARED((NP, D), jnp.float32),
            pltpu.SemaphoreType.DMA,
            pltpu.SemaphoreType.DMA,
            pltpu.SemaphoreType.DMA,
            pltpu.SemaphoreType.DMA,
            pltpu.SemaphoreType.DMA,
            pltpu.SemaphoreType.DMA,
        ],
    )
    def k(h_hbm, src_hbm, dst_hbm, out_hbm, src_v, didx_v, rows_v,
          acc_sh, g0, g1, i0, i1, s0, s1):
        c = lax.axis_index("c")
        s = lax.axis_index("s")
        gsem = (g0, g1)
        isem = (i0, i1)
        ssem = (s0, s1)
        start, cnt = _tile_range(c, s, tc0, tc1)
        pltpu.sync_copy(src_hbm.at[pl.ds(start, tmax)], src_v)

        @pl.loop(0, CHUNK)
        def _zb(i):
            for q in range(D // LANES):
                rows_v[0, i, pl.ds(q * LANES, LANES)] = jnp.zeros(
                    (LANES,), jnp.float32)

        base = s * ROWS_PER_SUB

        @pl.loop(0, ROWS_PER_SUB, step=CHUNK)
        def _za(r):
            pltpu.sync_copy(rows_v.at[0], acc_sh.at[pl.ds(base + r, CHUNK)])

        plsc.subcore_barrier()

        pltpu.async_copy(dst_hbm.at[start], didx_v.at[0], isem[0])
        pltpu.async_copy(h_hbm.at[src_v.at[0]], rows_v.at[0], gsem[0])

        @pl.loop(0, cnt, step=2)
        def _edges(j):
            for b in range(2):
                jj = j + b
                nb = b ^ 1

                @pl.when(jj + 1 < cnt)
                def _prefetch():
                    @pl.when(jj >= 1)
                    def _wait_sc():
                        pltpu.make_async_copy(
                            rows_v.at[nb], acc_sh.at[didx_v.at[nb]],
                            ssem[nb]).wait()
                    pltpu.async_copy(dst_hbm.at[start + jj + 1],
                                     didx_v.at[nb], isem[nb])
                    pltpu.async_copy(h_hbm.at[src_v.at[jj + 1]],
                                     rows_v.at[nb], gsem[nb])

                pltpu.make_async_copy(h_hbm.at[src_v.at[jj]], rows_v.at[b],
                                      gsem[b]).wait()
                pltpu.make_async_copy(dst_hbm.at[start], didx_v.at[b],
                                      isem[b]).wait()
                pltpu.async_copy(rows_v.at[b], acc_sh.at[didx_v.at[b]],
                                 ssem[b], add=True)

        pltpu.make_async_copy(rows_v.at[0], acc_sh.at[didx_v.at[0]],
                              ssem[0]).wait()
        pltpu.make_async_copy(rows_v.at[1], acc_sh.at[didx_v.at[1]],
                              ssem[1]).wait()
        plsc.subcore_barrier()
        pltpu.sync_copy(acc_sh.at[pl.ds(base, ROWS_PER_SUB)], out_hbm.at[c, s])

    out = k(h_pad, src_p, dst_p)
    return out.reshape(NCORE, NP, D)


def _sc_attention(z_pad, es_pad, ed_pad, src_p, dst_p, tc0, tc1):
    tmax = max(tc0, tc1)

    @functools.partial(
        pl.kernel, mesh=_mesh, compiler_params=_sc_params,
        out_type=(
            jax.ShapeDtypeStruct((NCORE, NSUB, ROWS_PER_SUB, D), jnp.float32),
            jax.ShapeDtypeStruct((NCORE, NSUB, ROWS_PER_SUB), jnp.float32),
        ),
        scratch_types=[
            pltpu.VMEM((tmax, CHUNK), jnp.int32),
            pltpu.VMEM((2, CHUNK), jnp.int32),
            pltpu.VMEM((CHUNK,), jnp.float32),
            pltpu.VMEM((CHUNK,), jnp.float32),
            pltpu.VMEM((CHUNK,), jnp.float32),
            pltpu.VMEM((CHUNK,), jnp.float32),
            pltpu.VMEM((2, CHUNK, D), jnp.float32),
            pltpu.VMEM_SHARED((NP, D), jnp.float32),
            pltpu.VMEM_SHARED((NP,), jnp.float32),
        ] + [pltpu.SemaphoreType.DMA] * 10,
    )
    def k(z_hbm, es_hbm, ed_hbm, src_hbm, dst_hbm, num_hbm, den_hbm,
          src_v, didx_v, wes0, wes1, wed0, wed1, rows_v, acc_sh, den_sh,
          g0, g1, e0, e1, f0, f1, s0, s1, d0, d1):
        c = lax.axis_index("c")
        s = lax.axis_index("s")
        gsem = (g0, g1)
        esem = (e0, e1)
        fsem = (f0, f1)
        ssem = (s0, s1)
        dsem = (d0, d1)
        wes = (wes0, wes1)
        wed = (wed0, wed1)
        start, cnt = _tile_range(c, s, tc0, tc1)
        pltpu.sync_copy(src_hbm.at[pl.ds(start, tmax)], src_v)

        @pl.loop(0, CHUNK)
        def _zb(i):
            for q in range(D // LANES):
                rows_v[0, i, pl.ds(q * LANES, LANES)] = jnp.zeros(
                    (LANES,), jnp.float32)

        @pl.loop(0, CHUNK, step=LANES)
        def _zw(i):
            wes0[pl.ds(i, LANES)] = jnp.zeros((LANES,), jnp.float32)

        base = s * ROWS_PER_SUB

        @pl.loop(0, ROWS_PER_SUB, step=CHUNK)
        def _za(r):
            pltpu.sync_copy(rows_v.at[0], acc_sh.at[pl.ds(base + r, CHUNK)])
            pltpu.sync_copy(wes0, den_sh.at[pl.ds(base + r, CHUNK)])

        plsc.subcore_barrier()

        pltpu.sync_copy(dst_hbm.at[start], didx_v.at[0])
        pltpu.async_copy(z_hbm.at[src_v.at[0]], rows_v.at[0], gsem[0])
        pltpu.async_copy(es_hbm.at[src_v.at[0]], wes[0], esem[0])
        pltpu.async_copy(ed_hbm.at[didx_v.at[0]], wed[0], fsem[0])

        @pl.loop(0, cnt, step=2)
        def _edges(j):
            for b in range(2):
                jj = j + b
                nb = b ^ 1

                @pl.when(jj + 1 < cnt)
                def _prefetch():
                    @pl.when(jj >= 1)
                    def _wait_reuse():
                        pltpu.make_async_copy(
                            rows_v.at[nb], acc_sh.at[didx_v.at[nb]],
                            ssem[nb]).wait()
                        pltpu.make_async_copy(
                            wes[nb], den_sh.at[didx_v.at[nb]],
                            dsem[nb]).wait()
                    pltpu.sync_copy(dst_hbm.at[start + jj + 1], didx_v.at[nb])
                    pltpu.async_copy(z_hbm.at[src_v.at[jj + 1]],
                                     rows_v.at[nb], gsem[nb])
                    pltpu.async_copy(es_hbm.at[src_v.at[jj + 1]], wes[nb],
                                     esem[nb])
                    pltpu.async_copy(ed_hbm.at[didx_v.at[nb]], wed[nb],
                                     fsem[nb])

                pltpu.make_async_copy(es_hbm.at[src_v.at[jj]], wes[b],
                                      esem[b]).wait()
                pltpu.make_async_copy(ed_hbm.at[didx_v.at[b]], wed[b],
                                      fsem[b]).wait()
                for q in range(CHUNK // LANES):
                    sl = pl.ds(q * LANES, LANES)
                    e = wes[b][sl] + wed[b][sl]
                    e = jnp.where(e > 0, e, 0.2 * e)
                    wes[b][sl] = jnp.exp(e)
                pltpu.async_copy(wes[b], den_sh.at[didx_v.at[b]], dsem[b],
                                 add=True)

                pltpu.make_async_copy(z_hbm.at[src_v.at[jj]], rows_v.at[b],
                                      gsem[b]).wait()

                @pl.loop(0, CHUNK)
                def _scale(ei):
                    wb = plsc.load_gather(wes[b],
                                          [jnp.broadcast_to(ei, (LANES,))])
                    for q in range(D // LANES):
                        sl = pl.ds(q * LANES, LANES)
                        rows_v[b, ei, sl] = rows_v[b, ei, sl] * wb

                pltpu.async_copy(rows_v.at[b], acc_sh.at[didx_v.at[b]],
                                 ssem[b], add=True)

        for b in range(2):
            pltpu.make_async_copy(rows_v.at[b], acc_sh.at[didx_v.at[b]],
                                  ssem[b]).wait()
            pltpu.make_async_copy(wes[b], den_sh.at[didx_v.at[b]],
                                  dsem[b]).wait()
        plsc.subcore_barrier()
        pltpu.sync_copy(acc_sh.at[pl.ds(base, ROWS_PER_SUB)], num_hbm.at[c, s])
        pltpu.sync_copy(den_sh.at[pl.ds(base, ROWS_PER_SUB)],
                        den_hbm.at[c, s])

    num, den = k(z_pad, es_pad, ed_pad, src_p, dst_p)
    return num.reshape(NCORE, NP, D), den.reshape(NCORE, NP)


BN = 2048


def _rows(pid):
    return pid * BN + lax.broadcasted_iota(jnp.int32, (BN, 1), 0)


def _ln_rows(t, g, b):
    mu = jnp.mean(t, axis=-1, keepdims=True)
    v = jnp.mean((t - mu) ** 2, axis=-1, keepdims=True)
    return (t - mu) / jnp.sqrt(v + 1e-5) * g + b


def _tc_rs(deg_t):
    def body(d_ref, o_ref):
        o_ref[...] = lax.rsqrt(
            jnp.sum(d_ref[...], axis=1, keepdims=True) + 1.0)

    return pl.pallas_call(
        body, out_shape=jax.ShapeDtypeStruct((NP, 1), jnp.float32))(deg_t)


def _tc_prescale(xp, W, b, rs):
    def body(x_ref, w_ref, b_ref, rs_ref, o_ref):
        h = jnp.dot(x_ref[...], w_ref[...],
                    preferred_element_type=jnp.float32) + b_ref[...]
        h = h * rs_ref[...]
        o_ref[...] = jnp.where(_rows(pl.program_id(0)) < N, h, 0.0)

    return pl.pallas_call(
        body,
        grid=(NP // BN,),
        in_specs=[pl.BlockSpec((BN, D), lambda i: (i, 0)),
                  pl.BlockSpec((D, D), lambda i: (0, 0)),
                  pl.BlockSpec((1, D), lambda i: (0, 0)),
                  pl.BlockSpec((BN, 1), lambda i: (i, 0))],
        out_specs=pl.BlockSpec((BN, D), lambda i: (i, 0)),
        out_shape=jax.ShapeDtypeStruct((NP, D), jnp.float32),
    )(xp, W, b.reshape(1, D), rs)


def _tc_conv_mid(p, rs, g, bb, W, b):
    def body(p_ref, rs_ref, g_ref, bb_ref, w_ref, b_ref, o_ref):
        rsb = rs_ref[...]
        t = (p_ref[0] + p_ref[1]) * rsb
        y = jax.nn.relu(_ln_rows(t, g_ref[...], bb_ref[...]))
        h = (jnp.dot(y, w_ref[...], preferred_element_type=jnp.float32)
             + b_ref[...]) * rsb
        o_ref[...] = jnp.where(_rows(pl.program_id(0)) < N, h, 0.0)

    return pl.pallas_call(
        body,
        grid=(NP // BN,),
        in_specs=[pl.BlockSpec((NCORE, BN, D), lambda i: (0, i, 0)),
                  pl.BlockSpec((BN, 1), lambda i: (i, 0)),
                  pl.BlockSpec((1, D), lambda i: (0, 0)),
                  pl.BlockSpec((1, D), lambda i: (0, 0)),
                  pl.BlockSpec((D, D), lambda i: (0, 0)),
                  pl.BlockSpec((1, D), lambda i: (0, 0))],
        out_specs=pl.BlockSpec((BN, D), lambda i: (i, 0)),
        out_shape=jax.ShapeDtypeStruct((NP, D), jnp.float32),
    )(p, rs, g.reshape(1, D), bb.reshape(1, D), W, b.reshape(1, D))


def _tc_att_prep(p, rs, g, bb, W_att, a_cat):
    def body(p_ref, rs_ref, g_ref, bb_ref, watt_ref, a_ref,
             z0_ref, z1_ref, z2_ref, z3_ref, e8_ref):
        t = (p_ref[0] + p_ref[1]) * rs_ref[...]
        y = jax.nn.relu(_ln_rows(t, g_ref[...], bb_ref[...]))
        y = jnp.where(_rows(pl.program_id(0)) < N, y, 0.0)
        z_refs = [z0_ref, z1_ref, z2_ref, z3_ref]
        cols = []
        for h in range(H):
            z = jnp.dot(y, watt_ref[h], preferred_element_type=jnp.float32)
            z_refs[h][...] = z
            cols.append(jnp.dot(z, a_ref[h],
                                preferred_element_type=jnp.float32))
        e8_ref[...] = jnp.concatenate(cols, axis=-1)

    zs = jax.ShapeDtypeStruct((NP, D), jnp.float32)
    return pl.pallas_call(
        body,
        grid=(NP // BN,),
        in_specs=[pl.BlockSpec((NCORE, BN, D), lambda i: (0, i, 0)),
                  pl.BlockSpec((BN, 1), lambda i: (i, 0)),
                  pl.BlockSpec((1, D), lambda i: (0, 0)),
                  pl.BlockSpec((1, D), lambda i: (0, 0)),
                  pl.BlockSpec((H, D, D), lambda i: (0, 0, 0)),
                  pl.BlockSpec((H, D, 2), lambda i: (0, 0, 0))],
        out_specs=[pl.BlockSpec((BN, D), lambda i: (i, 0))] * H
        + [pl.BlockSpec((BN, 2 * H), lambda i: (i, 0))],
        out_shape=[zs] * H
        + [jax.ShapeDtypeStruct((NP, 2 * H), jnp.float32)],
    )(p, rs, g.reshape(1, D), bb.reshape(1, D), W_att, a_cat)


def _tc_final(nums, den_ts, g2, b2, Wp, bp):
    def body(n0, n1, n2, n3, d0, d1, d2, d3, g_ref, bb_ref, w_ref, b_ref,
             o_ref):
        hs = []
        for n_ref, d_ref in zip((n0, n1, n2, n3), (d0, d1, d2, d3)):
            den = jnp.sum(d_ref[...], axis=1, keepdims=True) + 1e-9
            hs.append((n_ref[0] + n_ref[1]) / den)
        hcat = jnp.concatenate(hs, axis=-1)
        y = _ln_rows(hcat, g_ref[...], bb_ref[...])
        y = jnp.dot(y, w_ref[...], preferred_element_type=jnp.float32)
        o_ref[...] = jax.nn.relu(y + b_ref[...])

    hd = 2 * H * D
    return pl.pallas_call(
        body,
        grid=(NP // BN,),
        in_specs=[pl.BlockSpec((NCORE, BN, D), lambda i: (0, i, 0))] * H
        + [pl.BlockSpec((BN, NCORE), lambda i: (i, 0))] * H
        + [pl.BlockSpec((1, H * D), lambda i: (0, 0)),
           pl.BlockSpec((1, H * D), lambda i: (0, 0)),
           pl.BlockSpec((H * D, D), lambda i: (0, 0)),
           pl.BlockSpec((1, D), lambda i: (0, 0))],
        out_specs=pl.BlockSpec((BN, D), lambda i: (i, 0)),
        out_shape=jax.ShapeDtypeStruct((NP, D), jnp.float32),
    )(*nums, *den_ts, g2.reshape(1, H * D), b2.reshape(1, H * D), Wp,
      bp.reshape(1, D))


def kernel(x, edge_index, W_conv, b_conv, ln_g, ln_b, ln2_g, ln2_b, W_att,
           a_src, a_dst, W_proj, b_proj):
    src = edge_index[0]
    dst = edge_index[1]
    e = src.shape[0]
    cc0, cc1 = _split_counts(e, F_SLOW_CONV)
    ac0, ac1 = _split_counts(e, F_SLOW_ATT)
    src_c, dst_c = _pad_edge_arrays(src, dst, cc0, cc1)
    src_a, dst_a = _pad_edge_arrays(src, dst, ac0, ac1)

    deg_part = _sc_degree(dst_c, cc0, cc1)
    rs = _tc_rs(deg_part.T)

    xp = jnp.pad(x, ((0, NP - N), (0, 0)))
    h = _tc_prescale(xp, W_conv[0], b_conv[0], rs)
    p = _sc_segsum(h, src_c, dst_c, cc0, cc1)
    h = _tc_conv_mid(p, rs, ln_g, ln_b, W_conv[1], b_conv[1])
    p = _sc_segsum(h, src_c, dst_c, cc0, cc1)

    a_cat = jnp.stack([a_src, a_dst], axis=-1)
    *zs, e8 = _tc_att_prep(p, rs, ln_g, ln_b, W_att, a_cat)

    nums, den_ts = [], []
    for hh in range(H):
        num, den = _sc_attention(zs[hh], e8[:, 2 * hh], e8[:, 2 * hh + 1],
                                 src_a, dst_a, ac0, ac1)
        nums.append(num)
        den_ts.append(den.T)
    out = _tc_final(nums, den_ts, ln2_g, ln2_b, W_proj, b_proj)
    return out[:N]
